# baseline (speedup 1.0000x reference)
import functools

import jax
import jax.numpy as jnp
from jax import lax
from jax.experimental import pallas as pl
from jax.experimental.pallas import tpu as pltpu
from jax.experimental.pallas import tpu_sc as plsc

E = 8
R = 64
DIN = 4096
DOUT = 4096
SCALING = 16 / 64
NTOK = 8192

T = 256
NT = NTOK // T
J = NT + E - 1

_NSUB = 16
_CH = NTOK // _NSUB

_NW = 32
_GCH = NTOK // _NW


_GW = DIN // 2


def _router_body(w_ref, x_ref, out_ref, xb_ref):
    xv = x_ref[...]
    out_ref[...] = lax.dot_general(
        w_ref[...], xv, (((1,), (1,)), ((), ())),
        preferred_element_type=jnp.float32)
    bits = lax.bitcast_convert_type(xv.astype(jnp.bfloat16), jnp.uint16)
    lo = bits[:, :_GW].astype(jnp.uint32)
    hi = bits[:, _GW:].astype(jnp.uint32)
    xb_ref[...] = lax.bitcast_convert_type(
        lo | (hi << jnp.uint32(16)), jnp.int32)


def _router(x2d, w_route):
    tblk = 512
    return pl.pallas_call(
        _router_body,
        grid=(NTOK // tblk,),
        in_specs=[pl.BlockSpec((E, DIN), lambda t: (0, 0)),
                  pl.BlockSpec((tblk, DIN), lambda t: (t, 0))],
        out_specs=[pl.BlockSpec((E, tblk), lambda t: (0, t)),
                   pl.BlockSpec((tblk, _GW), lambda t: (t, 0))],
        out_shape=[jax.ShapeDtypeStruct((E, NTOK), jnp.float32),
                   jax.ShapeDtypeStruct((NTOK, _GW), jnp.int32)],
    )(w_route, x2d)


def _route_sc(tmpT):
    mesh = plsc.VectorSubcoreMesh(
        core_axis_name="c", subcore_axis_name="s", num_cores=1)

    @functools.partial(
        pl.kernel,
        out_type=(
            jax.ShapeDtypeStruct((_NSUB, 4, 128), jnp.int32),
            jax.ShapeDtypeStruct((NTOK,), jnp.int32),
            jax.ShapeDtypeStruct((NTOK,), jnp.float32),
            jax.ShapeDtypeStruct((16,), jnp.int32),
            jax.ShapeDtypeStruct((_NSUB, 16), jnp.int32),
        ),
        mesh=mesh,
        scratch_types=[
            pltpu.VMEM((E, _CH), jnp.float32),
            pltpu.VMEM((_CH,), jnp.int32),
            pltpu.VMEM((_CH,), jnp.float32),
            pltpu.VMEM((4, 128), jnp.int32),
            pltpu.VMEM((_CH,), jnp.int32),
            pltpu.VMEM((16,), jnp.int32),
            pltpu.VMEM((_NSUB, 16), jnp.int32),
            pltpu.VMEM((16,), jnp.int32),
            pltpu.VMEM_SHARED((NTOK,), jnp.float32),
            pltpu.VMEM_SHARED((NTOK,), jnp.int32),
        ],
        compiler_params=pltpu.CompilerParams(needs_layout_passes=False),
    )
    def k(tmpT_hbm, dest_hbm, perm_hbm, gs_hbm, counts_hbm, histg_hbm,
          logit_v, eid_v, gate_v, dest_v, tok_v, hist_v, grid_v, cnt_v,
          gs_sh, perm_sh):
        wid = lax.axis_index("s")
        base = wid * _CH
        for e in range(E):
            pltpu.sync_copy(tmpT_hbm.at[e, pl.ds(base, _CH)], logit_v.at[e])

        runc = [jnp.int32(0)] * E
        for v in range(_CH // 16):
            sl = pl.ds(v * 16, 16)
            m = logit_v[0, sl]
            idx = jnp.zeros((16,), jnp.int32)
            for e in range(1, E):
                xv = logit_v[e, sl]
                upd = xv > m
                idx = jnp.where(upd, jnp.int32(e), idx)
                m = jnp.where(upd, xv, m)
            eid_v[sl] = idx
            gate_v[sl] = m * SCALING
            for e in range(E):
                runc[e] = runc[e] + jnp.sum((idx == e).astype(jnp.int32))
        lane = lax.broadcasted_iota(jnp.int32, (16,), 0)
        hist_vec = jnp.zeros((16,), jnp.int32)
        for e in range(E):
            hist_vec = jnp.where(lane == e, runc[e], hist_vec)
        hist_v[...] = hist_vec

        pltpu.sync_copy(hist_v, histg_hbm.at[wid])
        plsc.subcore_barrier()
        pltpu.sync_copy(histg_hbm, grid_v)

        tot_vec = jnp.zeros((16,), jnp.int32)
        before_vec = jnp.zeros((16,), jnp.int32)
        for w in range(_NSUB):
            rowv = grid_v[w]
            tot_vec = tot_vec + rowv
            mine = jnp.where(jnp.int32(w) < wid, jnp.int32(1), jnp.int32(0))
            before_vec = before_vec + rowv * mine
        offs_vec = plsc.cumsum(tot_vec) - tot_vec
        bases_vec = offs_vec + before_vec
        bases = [bases_vec[e] for e in range(E)]
        cnt_v[...] = tot_vec

        runc2 = [jnp.int32(0)] * E
        for v in range(_CH // 16):
            sl = pl.ds(v * 16, 16)
            idx = eid_v[sl]
            d = jnp.zeros((16,), jnp.int32)
            for e in range(E):
                mk = idx == e
                mi = mk.astype(jnp.int32)
                cs = plsc.cumsum(mi)
                dv = (bases[e] + runc2[e] - 1) + cs
                d = jnp.where(mk, dv, d)
                runc2[e] = runc2[e] + jnp.sum(mi)
            dest_v[v // 8, pl.ds((v % 8) * 16, 16)] = d
            tok_v[sl] = base + v * 16 + lax.broadcasted_iota(jnp.int32, (16,), 0)

        pltpu.sync_copy(dest_v, dest_hbm.at[wid])
        for jj in range(4):
            pltpu.sync_copy(gate_v.at[pl.ds(jj * 128, 128)],
                            gs_sh.at[dest_v.at[jj]])
            pltpu.sync_copy(tok_v.at[pl.ds(jj * 128, 128)],
                            perm_sh.at[dest_v.at[jj]])
        plsc.subcore_barrier()
        pltpu.sync_copy(gs_sh.at[pl.ds(base, _CH)], gs_hbm.at[pl.ds(base, _CH)])
        pltpu.sync_copy(perm_sh.at[pl.ds(base, _CH)],
                        perm_hbm.at[pl.ds(base, _CH)])

        @pl.when(wid == 0)
        def _():
            pltpu.sync_copy(cnt_v, counts_hbm)

    return k(tmpT)


def _gather_rows(src, idx, width, dtype, gb):
    mesh = plsc.VectorSubcoreMesh(core_axis_name="c", subcore_axis_name="s")

    @functools.partial(
        pl.kernel,
        out_type=jax.ShapeDtypeStruct((NTOK, width), dtype),
        mesh=mesh,
        scratch_types=[
            pltpu.VMEM((_GCH,), jnp.int32),
            pltpu.VMEM((2, gb, width), dtype),
            pltpu.SemaphoreType.DMA((2,)),
        ],
        compiler_params=pltpu.CompilerParams(needs_layout_passes=False),
    )
    def k(src_hbm, idx_hbm, out_hbm, idx_v, buf_v, sems):
        wid = lax.axis_index("c") * 16 + lax.axis_index("s")
        base = wid * _GCH
        pltpu.sync_copy(idx_hbm.at[wid], idx_v)
        nst = _GCH // gb

        def fire(j, slot):
            return pltpu.async_copy(
                src_hbm.at[idx_v.at[pl.ds(j * gb, gb)]],
                buf_v.at[slot], sems.at[slot])

        descs = [fire(0, 0), None]
        for j in range(nst):
            s = j % 2
            if j + 1 < nst:
                descs[1 - s] = fire(j + 1, 1 - s)
            descs[s].wait()
            pltpu.sync_copy(buf_v.at[s],
                            out_hbm.at[pl.ds(base + j * gb, gb)])

    return k(src, idx)


def _jobs(counts):
    counts = counts[:E]
    offs = jnp.concatenate(
        [jnp.zeros((1,), jnp.int32), jnp.cumsum(counts, dtype=jnp.int32)])
    t = jnp.arange(NT, dtype=jnp.int32)
    tlo = (t * T)[:, None]
    lo = jnp.maximum(offs[:-1][None, :], tlo)
    hi = jnp.minimum(offs[1:][None, :], tlo + T)
    valid = (hi > lo).reshape(-1)
    pos = jnp.cumsum(valid.astype(jnp.int32)) - 1
    posc = jnp.where(valid, pos, J)
    tt = jnp.broadcast_to(t[:, None], (NT, E)).reshape(-1)
    ee = jnp.broadcast_to(
        jnp.arange(E, dtype=jnp.int32)[None, :], (NT, E)).reshape(-1)
    job_t = jnp.full((J + 1,), NT - 1, jnp.int32).at[posc].set(tt)[:J]
    job_e = jnp.full((J + 1,), E - 1, jnp.int32).at[posc].set(ee)[:J]
    job_lo = jnp.zeros((J + 1,), jnp.int32).at[posc].set(
        (lo - tlo).reshape(-1))[:J]
    job_hi = jnp.zeros((J + 1,), jnp.int32).at[posc].set(
        (hi - tlo).reshape(-1))[:J]
    job_first = jnp.zeros((J + 1,), jnp.int32).at[posc].set(
        (lo == tlo).astype(jnp.int32).reshape(-1))[:J]
    return job_t, job_e, job_lo, job_hi, job_first


def _moe_body(tr, er, lor, hir, fr, xs_ref, a_ref, b_ref, gs_ref, out_ref):
    j = pl.program_id(0)
    lo = lor[j]
    hi = hir[j]
    first = fr[j]

    @pl.when(hi > lo)
    def _():
        wu = lax.bitcast_convert_type(xs_ref[...], jnp.uint32)
        xlo = lax.bitcast_convert_type(
            (wu & jnp.uint32(0xFFFF)).astype(jnp.uint16), jnp.bfloat16)
        xhi = lax.bitcast_convert_type(
            (wu >> jnp.uint32(16)).astype(jnp.uint16), jnp.bfloat16)
        ae = a_ref[0].astype(jnp.bfloat16)
        aT = lax.dot_general(
            ae[:, :_GW], xlo, (((1,), (1,)), ((), ())),
            preferred_element_type=jnp.float32)
        aT += lax.dot_general(
            ae[:, _GW:], xhi, (((1,), (1,)), ((), ())),
            preferred_element_type=jnp.float32)
        cols = lax.broadcasted_iota(jnp.int32, (1, T), 1)
        sel = jnp.where((cols >= lo) & (cols < hi), gs_ref[0], 0.0)
        aTs = (aT * sel).astype(jnp.bfloat16)
        y = lax.dot_general(
            aTs, b_ref[0].astype(jnp.bfloat16), (((0,), (1,)), ((), ())),
            preferred_element_type=jnp.float32)

        @pl.when(first == 1)
        def _():
            out_ref[...] = y

        @pl.when(first == 0)
        def _():
            out_ref[...] += y


def _grouped(xs, A, Bw, gs2d, jobs):
    grid_spec = pltpu.PrefetchScalarGridSpec(
        num_scalar_prefetch=5,
        grid=(J,),
        in_specs=[
            pl.BlockSpec((T, _GW), lambda j, tr, er, lor, hir, fr: (tr[j], 0)),
            pl.BlockSpec((1, R, DIN),
                         lambda j, tr, er, lor, hir, fr: (er[j], 0, 0)),
            pl.BlockSpec((1, DOUT, R),
                         lambda j, tr, er, lor, hir, fr: (er[j], 0, 0)),
            pl.BlockSpec((1, 1, T),
                         lambda j, tr, er, lor, hir, fr: (tr[j], 0, 0)),
        ],
        out_specs=pl.BlockSpec((T, DOUT),
                               lambda j, tr, er, lor, hir, fr: (tr[j], 0)),
    )
    return pl.pallas_call(
        _moe_body,
        grid_spec=grid_spec,
        out_shape=jax.ShapeDtypeStruct((NTOK, DOUT), jnp.float32),
        compiler_params=pltpu.CompilerParams(
            dimension_semantics=("arbitrary",)),
    )(*jobs, xs, A, Bw, gs2d)


def kernel(x, w_route, A, Bw):
    x2d = x.reshape(NTOK, DIN)
    tmpT, xb = _router(x2d, w_route)
    dest3, perm, gs, counts, _histg = _route_sc(tmpT)
    xs = _gather_rows(xb, perm.reshape(_NW, _GCH), _GW, jnp.int32, 16)
    jobs = _jobs(counts)
    ys = _grouped(xs, A, Bw, gs.reshape(NT, 1, T), jobs)
    out = _gather_rows(ys, dest3.reshape(_NW, _GCH), DIN, jnp.float32, 8)
    return out.reshape(x.shape[0], x.shape[1], DOUT)


# speedup vs baseline: 1.0134x; 1.0134x over previous
import functools

import jax
import jax.numpy as jnp
from jax import lax
from jax.experimental import pallas as pl
from jax.experimental.pallas import tpu as pltpu
from jax.experimental.pallas import tpu_sc as plsc

E = 8
R = 64
DIN = 4096
DOUT = 4096
SCALING = 16 / 64
NTOK = 8192

T = 256
NT = NTOK // T
J = NT + E - 1

_NSUB = 16
_CH = NTOK // _NSUB

_NW = 32
_GCH = NTOK // _NW


_GW = DIN // 2


def _router_body(w_ref, x_ref, out_ref, xb_ref):
    xv = x_ref[...]
    out_ref[...] = lax.dot_general(
        w_ref[...], xv, (((1,), (1,)), ((), ())),
        preferred_element_type=jnp.float32)
    bits = lax.bitcast_convert_type(xv.astype(jnp.bfloat16), jnp.uint16)
    lo = bits[:, :_GW].astype(jnp.uint32)
    hi = bits[:, _GW:].astype(jnp.uint32)
    xb_ref[...] = lax.bitcast_convert_type(
        lo | (hi << jnp.uint32(16)), jnp.int32)


def _router(x2d, w_route):
    tblk = 512
    return pl.pallas_call(
        _router_body,
        grid=(NTOK // tblk,),
        in_specs=[pl.BlockSpec((E, DIN), lambda t: (0, 0)),
                  pl.BlockSpec((tblk, DIN), lambda t: (t, 0))],
        out_specs=[pl.BlockSpec((E, tblk), lambda t: (0, t)),
                   pl.BlockSpec((tblk, _GW), lambda t: (t, 0))],
        out_shape=[jax.ShapeDtypeStruct((E, NTOK), jnp.float32),
                   jax.ShapeDtypeStruct((NTOK, _GW), jnp.int32)],
    )(w_route, x2d)


def _route_sc(tmpT):
    mesh = plsc.VectorSubcoreMesh(
        core_axis_name="c", subcore_axis_name="s", num_cores=1)

    @functools.partial(
        pl.kernel,
        out_type=(
            jax.ShapeDtypeStruct((_NSUB, 4, 128), jnp.int32),
            jax.ShapeDtypeStruct((NTOK,), jnp.int32),
            jax.ShapeDtypeStruct((NTOK,), jnp.float32),
            jax.ShapeDtypeStruct((16,), jnp.int32),
            jax.ShapeDtypeStruct((_NSUB, 16), jnp.int32),
        ),
        mesh=mesh,
        scratch_types=[
            pltpu.VMEM((E, _CH), jnp.float32),
            pltpu.VMEM((_CH,), jnp.int32),
            pltpu.VMEM((_CH,), jnp.float32),
            pltpu.VMEM((4, 128), jnp.int32),
            pltpu.VMEM((_CH,), jnp.int32),
            pltpu.VMEM((16,), jnp.int32),
            pltpu.VMEM((_NSUB, 16), jnp.int32),
            pltpu.VMEM((16,), jnp.int32),
            pltpu.VMEM_SHARED((NTOK,), jnp.float32),
            pltpu.VMEM_SHARED((NTOK,), jnp.int32),
        ],
        compiler_params=pltpu.CompilerParams(needs_layout_passes=False),
    )
    def k(tmpT_hbm, dest_hbm, perm_hbm, gs_hbm, counts_hbm, histg_hbm,
          logit_v, eid_v, gate_v, dest_v, tok_v, hist_v, grid_v, cnt_v,
          gs_sh, perm_sh):
        wid = lax.axis_index("s")
        base = wid * _CH
        for e in range(E):
            pltpu.sync_copy(tmpT_hbm.at[e, pl.ds(base, _CH)], logit_v.at[e])

        runc = [jnp.int32(0)] * E
        for v in range(_CH // 16):
            sl = pl.ds(v * 16, 16)
            m = logit_v[0, sl]
            idx = jnp.zeros((16,), jnp.int32)
            for e in range(1, E):
                xv = logit_v[e, sl]
                upd = xv > m
                idx = jnp.where(upd, jnp.int32(e), idx)
                m = jnp.where(upd, xv, m)
            eid_v[sl] = idx
            gate_v[sl] = m * SCALING
            for e in range(E):
                runc[e] = runc[e] + jnp.sum((idx == e).astype(jnp.int32))
        lane = lax.broadcasted_iota(jnp.int32, (16,), 0)
        hist_vec = jnp.zeros((16,), jnp.int32)
        for e in range(E):
            hist_vec = jnp.where(lane == e, runc[e], hist_vec)
        hist_v[...] = hist_vec

        pltpu.sync_copy(hist_v, histg_hbm.at[wid])
        plsc.subcore_barrier()
        pltpu.sync_copy(histg_hbm, grid_v)

        tot_vec = jnp.zeros((16,), jnp.int32)
        before_vec = jnp.zeros((16,), jnp.int32)
        for w in range(_NSUB):
            rowv = grid_v[w]
            tot_vec = tot_vec + rowv
            mine = jnp.where(jnp.int32(w) < wid, jnp.int32(1), jnp.int32(0))
            before_vec = before_vec + rowv * mine
        offs_vec = plsc.cumsum(tot_vec) - tot_vec
        bases_vec = offs_vec + before_vec
        bases = [bases_vec[e] for e in range(E)]
        cnt_v[...] = tot_vec

        runc2 = [jnp.int32(0)] * E
        for v in range(_CH // 16):
            sl = pl.ds(v * 16, 16)
            idx = eid_v[sl]
            d = jnp.zeros((16,), jnp.int32)
            for e in range(E):
                mk = idx == e
                mi = mk.astype(jnp.int32)
                cs = plsc.cumsum(mi)
                dv = (bases[e] + runc2[e] - 1) + cs
                d = jnp.where(mk, dv, d)
                runc2[e] = runc2[e] + jnp.sum(mi)
            dest_v[v // 8, pl.ds((v % 8) * 16, 16)] = d
            tok_v[sl] = base + v * 16 + lax.broadcasted_iota(jnp.int32, (16,), 0)

        pltpu.sync_copy(dest_v, dest_hbm.at[wid])
        for jj in range(4):
            pltpu.sync_copy(gate_v.at[pl.ds(jj * 128, 128)],
                            gs_sh.at[dest_v.at[jj]])
            pltpu.sync_copy(tok_v.at[pl.ds(jj * 128, 128)],
                            perm_sh.at[dest_v.at[jj]])
        plsc.subcore_barrier()
        pltpu.sync_copy(gs_sh.at[pl.ds(base, _CH)], gs_hbm.at[pl.ds(base, _CH)])
        pltpu.sync_copy(perm_sh.at[pl.ds(base, _CH)],
                        perm_hbm.at[pl.ds(base, _CH)])

        @pl.when(wid == 0)
        def _():
            pltpu.sync_copy(cnt_v, counts_hbm)

    return k(tmpT)


def _gather_rows(src, idx, width, dtype, gb):
    mesh = plsc.VectorSubcoreMesh(core_axis_name="c", subcore_axis_name="s")

    @functools.partial(
        pl.kernel,
        out_type=jax.ShapeDtypeStruct((NTOK, width), dtype),
        mesh=mesh,
        scratch_types=[
            pltpu.VMEM((_GCH,), jnp.int32),
            pltpu.VMEM((2, gb, width), dtype),
            pltpu.SemaphoreType.DMA((2,)),
        ],
        compiler_params=pltpu.CompilerParams(needs_layout_passes=False),
    )
    def k(src_hbm, idx_hbm, out_hbm, idx_v, buf_v, sems):
        wid = lax.axis_index("c") * 16 + lax.axis_index("s")
        base = wid * _GCH
        pltpu.sync_copy(idx_hbm.at[wid], idx_v)
        nst = _GCH // gb

        def fire(j, slot):
            return pltpu.async_copy(
                src_hbm.at[idx_v.at[pl.ds(j * gb, gb)]],
                buf_v.at[slot], sems.at[slot])

        descs = [fire(0, 0), None]
        for j in range(nst):
            s = j % 2
            if j + 1 < nst:
                descs[1 - s] = fire(j + 1, 1 - s)
            descs[s].wait()
            pltpu.sync_copy(buf_v.at[s],
                            out_hbm.at[pl.ds(base + j * gb, gb)])

    return k(src, idx)


def _jobs(counts):
    counts = counts[:E]
    offs = jnp.concatenate(
        [jnp.zeros((1,), jnp.int32), jnp.cumsum(counts, dtype=jnp.int32)])
    t = jnp.arange(NT, dtype=jnp.int32)
    tlo = (t * T)[:, None]
    lo = jnp.maximum(offs[:-1][None, :], tlo)
    hi = jnp.minimum(offs[1:][None, :], tlo + T)
    valid = (hi > lo).reshape(-1)
    pos = jnp.cumsum(valid.astype(jnp.int32)) - 1
    posc = jnp.where(valid, pos, J)
    tt = jnp.broadcast_to(t[:, None], (NT, E)).reshape(-1)
    ee = jnp.broadcast_to(
        jnp.arange(E, dtype=jnp.int32)[None, :], (NT, E)).reshape(-1)
    job_t = jnp.full((J + 1,), NT - 1, jnp.int32).at[posc].set(tt)[:J]
    job_e = jnp.full((J + 1,), E - 1, jnp.int32).at[posc].set(ee)[:J]
    job_lo = jnp.zeros((J + 1,), jnp.int32).at[posc].set(
        (lo - tlo).reshape(-1))[:J]
    job_hi = jnp.zeros((J + 1,), jnp.int32).at[posc].set(
        (hi - tlo).reshape(-1))[:J]
    job_first = jnp.zeros((J + 1,), jnp.int32).at[posc].set(
        (lo == tlo).astype(jnp.int32).reshape(-1))[:J]
    return job_t, job_e, job_lo, job_hi, job_first


def _moe_body(tr, er, lor, hir, fr, xs_ref, a_ref, b_ref, gs_ref, out_ref):
    j = pl.program_id(0)
    lo = lor[j]
    hi = hir[j]
    first = fr[j]

    @pl.when(hi > lo)
    def _():
        wu = lax.bitcast_convert_type(xs_ref[...], jnp.uint32)
        xlo = lax.bitcast_convert_type(
            (wu & jnp.uint32(0xFFFF)).astype(jnp.uint16), jnp.bfloat16)
        xhi = lax.bitcast_convert_type(
            (wu >> jnp.uint32(16)).astype(jnp.uint16), jnp.bfloat16)
        ae = a_ref[0]
        aT = lax.dot_general(
            ae[:, :_GW], xlo, (((1,), (1,)), ((), ())),
            preferred_element_type=jnp.float32)
        aT += lax.dot_general(
            ae[:, _GW:], xhi, (((1,), (1,)), ((), ())),
            preferred_element_type=jnp.float32)
        cols = lax.broadcasted_iota(jnp.int32, (1, T), 1)
        sel = jnp.where((cols >= lo) & (cols < hi), gs_ref[0], 0.0)
        aTs = (aT * sel).astype(jnp.bfloat16)
        y = lax.dot_general(
            aTs, b_ref[0], (((0,), (1,)), ((), ())),
            preferred_element_type=jnp.float32)

        @pl.when(first == 1)
        def _():
            out_ref[...] = y

        @pl.when(first == 0)
        def _():
            out_ref[...] += y


def _grouped(xs, A, Bw, gs2d, jobs):
    grid_spec = pltpu.PrefetchScalarGridSpec(
        num_scalar_prefetch=5,
        grid=(J,),
        in_specs=[
            pl.BlockSpec((T, _GW), lambda j, tr, er, lor, hir, fr: (tr[j], 0)),
            pl.BlockSpec((1, R, DIN),
                         lambda j, tr, er, lor, hir, fr: (er[j], 0, 0)),
            pl.BlockSpec((1, DOUT, R),
                         lambda j, tr, er, lor, hir, fr: (er[j], 0, 0)),
            pl.BlockSpec((1, 1, T),
                         lambda j, tr, er, lor, hir, fr: (tr[j], 0, 0)),
        ],
        out_specs=pl.BlockSpec((T, DOUT),
                               lambda j, tr, er, lor, hir, fr: (tr[j], 0)),
    )
    return pl.pallas_call(
        _moe_body,
        grid_spec=grid_spec,
        out_shape=jax.ShapeDtypeStruct((NTOK, DOUT), jnp.float32),
        compiler_params=pltpu.CompilerParams(
            dimension_semantics=("arbitrary",)),
    )(*jobs, xs, A, Bw, gs2d)


def kernel(x, w_route, A, Bw):
    x2d = x.reshape(NTOK, DIN)
    tmpT, xb = _router(x2d, w_route)
    dest3, perm, gs, counts, _histg = _route_sc(tmpT)
    xs = _gather_rows(xb, perm.reshape(_NW, _GCH), _GW, jnp.int32, 16)
    jobs = _jobs(counts)
    ys = _grouped(xs, A.astype(jnp.bfloat16), Bw.astype(jnp.bfloat16),
                  gs.reshape(NT, 1, T), jobs)
    out = _gather_rows(ys, dest3.reshape(_NW, _GCH), DIN, jnp.float32, 8)
    return out.reshape(x.shape[0], x.shape[1], DOUT)
